# baseline (speedup 1.0000x reference)
import jax
import jax.numpy as jnp
from jax import lax
from jax.experimental import pallas as pl
from jax.experimental.pallas import tpu as pltpu
from jax.experimental.pallas import tpu_sc as plsc

N = 320000
D = 128
S = 512
NC = 2
NS = 16
NW = NC * NS
RPW = N // NW
K = 100
G = RPW // K
CHUNK = 200
GPC = CHUNK // K
NCHUNK = RPW // CHUNK
ROWS_PER_TILE_INIT = S // NS
HB = 2000
HG = N // HB


def _sc_segment_sum(H_v, bidx, zacc):
    mesh = plsc.VectorSubcoreMesh(core_axis_name="c", subcore_axis_name="s")

    def body(hv, idx_hbm, zacc_hbm, psum, idx_v, buf0, buf1, acc_sh,
             sem0, sem1, semS):
        cid = lax.axis_index("c")
        sid = lax.axis_index("s")
        wid = cid * NS + sid
        row_base = wid * RPW

        pltpu.sync_copy(idx_hbm.at[wid], idx_v)
        r0 = sid * ROWS_PER_TILE_INIT
        pltpu.sync_copy(zacc_hbm.at[pl.ds(r0, ROWS_PER_TILE_INIT)],
                        acc_sh.at[pl.ds(r0, ROWS_PER_TILE_INIT)])
        plsc.subcore_barrier()

        def hv_chunk(c):
            return hv.at[pl.ds(row_base + c * CHUNK, CHUNK)]

        def issue(c, buf, sem):
            pltpu.async_copy(hv_chunk(c), buf, sem)

        def wait(c, buf, sem):
            pltpu.make_async_copy(hv_chunk(c), buf, sem).wait()

        def scatter(c, buf):
            for g in range(GPC):
                jg = c * GPC + g
                pltpu.async_copy(buf.at[pl.ds(g * K, K)],
                                 acc_sh.at[idx_v.at[jg]], semS, add=True)
                pltpu.make_async_copy(
                    buf.at[pl.ds(g * K, K)], acc_sh.at[idx_v.at[jg]],
                    semS).wait()

        issue(0, buf0, sem0)

        def pair_body(j, carry):
            c0 = 2 * j
            wait(c0, buf0, sem0)
            issue(c0 + 1, buf1, sem1)
            scatter(c0, buf0)
            wait(c0 + 1, buf1, sem1)
            issue(c0 + 2, buf0, sem0)
            scatter(c0 + 1, buf1)
            return carry

        lax.fori_loop(0, (NCHUNK - 1) // 2, pair_body, 0)
        if NCHUNK % 2:
            wait(NCHUNK - 1, buf0, sem0)
            scatter(NCHUNK - 1, buf0)
        else:
            wait(NCHUNK - 2, buf0, sem0)
            issue(NCHUNK - 1, buf1, sem1)
            scatter(NCHUNK - 2, buf0)
            wait(NCHUNK - 1, buf1, sem1)
            scatter(NCHUNK - 1, buf1)
        plsc.subcore_barrier()

        pltpu.sync_copy(acc_sh.at[pl.ds(r0, ROWS_PER_TILE_INIT)],
                        psum.at[cid, pl.ds(r0, ROWS_PER_TILE_INIT)])

    fn = pl.kernel(
        body,
        out_type=jax.ShapeDtypeStruct((NC, S, D), jnp.float32),
        mesh=mesh,
        scratch_types=(
            pltpu.VMEM((G, K), jnp.int32),
            pltpu.VMEM((CHUNK, D), jnp.float32),
            pltpu.VMEM((CHUNK, D), jnp.float32),
            pltpu.VMEM_SHARED((S, D), jnp.float32),
            pltpu.SemaphoreType.DMA,
            pltpu.SemaphoreType.DMA,
            pltpu.SemaphoreType.DMA,
        ),
    )
    return fn(H_v, bidx, zacc)


def _hist_body(b_ref, out_ref):
    ids = b_ref[...]
    q = ids // 128
    r = ids - q * 128
    q_oh = (q == lax.broadcasted_iota(jnp.int32, (HB, 4), 1)
            ).astype(jnp.float32)
    r_oh = (r == lax.broadcasted_iota(jnp.int32, (HB, 128), 1)
            ).astype(jnp.float32)
    cmat = lax.dot_general(q_oh, r_oh, (((0,), (0,)), ((), ())),
                           preferred_element_type=jnp.float32)
    acc = jnp.concatenate([cmat, jnp.zeros((4, 128), jnp.float32)], axis=0)

    @pl.when(pl.program_id(0) == 0)
    def _init():
        out_ref[...] = jnp.zeros((8, 128), jnp.float32)

    out_ref[...] += acc


def _segment_counts(batch):
    bcol = batch.reshape(N, 1)
    hist = pl.pallas_call(
        _hist_body,
        grid=(HG,),
        in_specs=[pl.BlockSpec((HB, 1), lambda i: (i, 0))],
        out_specs=pl.BlockSpec((8, 128), lambda i: (0, 0)),
        out_shape=jax.ShapeDtypeStruct((8, 128), jnp.float32),
    )(bcol)
    return hist[0:4].reshape(S, 1)


def _finalize_body(ps_ref, pc_ref, out_ref):
    sums = ps_ref[0] + ps_ref[1]
    counts = jnp.maximum(pc_ref[...], 1.0)
    out_ref[...] = sums / counts


def kernel(H_v, batch):
    bidx = batch.reshape(NW, G, K)
    zacc = jnp.zeros((S, D), jnp.float32)
    psum = _sc_segment_sum(H_v, bidx, zacc)
    counts = _segment_counts(batch)
    return pl.pallas_call(
        _finalize_body,
        out_shape=jax.ShapeDtypeStruct((S, D), jnp.float32),
    )(psum, counts)


# speedup vs baseline: 1.0890x; 1.0890x over previous
import jax
import jax.numpy as jnp
from jax import lax
from jax.experimental import pallas as pl
from jax.experimental.pallas import tpu as pltpu
from jax.experimental.pallas import tpu_sc as plsc

N = 320000
D = 128
S = 512
NC = 2
NS = 16
NW = NC * NS
RPW = N // NW
K = 100
G = RPW // K
CHUNK = 200
GPC = CHUNK // K
NCHUNK = RPW // CHUNK
ROWS_PER_TILE_INIT = S // NS
HB = 8000
HG = N // HB


def _sc_segment_sum(H_v, bidx, zacc):
    mesh = plsc.VectorSubcoreMesh(core_axis_name="c", subcore_axis_name="s")

    def body(hv, idx_hbm, zacc_hbm, psum, idx_v, buf0, buf1, acc_sh,
             sem0, sem1, semS):
        cid = lax.axis_index("c")
        sid = lax.axis_index("s")
        wid = cid * NS + sid
        row_base = wid * RPW

        pltpu.sync_copy(idx_hbm.at[wid], idx_v)
        r0 = sid * ROWS_PER_TILE_INIT
        pltpu.sync_copy(zacc_hbm.at[pl.ds(r0, ROWS_PER_TILE_INIT)],
                        acc_sh.at[pl.ds(r0, ROWS_PER_TILE_INIT)])
        plsc.subcore_barrier()

        def hv_chunk(c):
            return hv.at[pl.ds(row_base + c * CHUNK, CHUNK)]

        def issue(c, buf, sem):
            pltpu.async_copy(hv_chunk(c), buf, sem)

        def wait(c, buf, sem):
            pltpu.make_async_copy(hv_chunk(c), buf, sem).wait()

        def scatter(c, buf):
            for g in range(GPC):
                jg = c * GPC + g
                pltpu.async_copy(buf.at[pl.ds(g * K, K)],
                                 acc_sh.at[idx_v.at[jg]], semS, add=True)
                pltpu.make_async_copy(
                    buf.at[pl.ds(g * K, K)], acc_sh.at[idx_v.at[jg]],
                    semS).wait()

        issue(0, buf0, sem0)

        def pair_body(j, carry):
            c0 = 2 * j
            wait(c0, buf0, sem0)
            issue(c0 + 1, buf1, sem1)
            scatter(c0, buf0)
            wait(c0 + 1, buf1, sem1)
            issue(c0 + 2, buf0, sem0)
            scatter(c0 + 1, buf1)
            return carry

        lax.fori_loop(0, (NCHUNK - 1) // 2, pair_body, 0)
        if NCHUNK % 2:
            wait(NCHUNK - 1, buf0, sem0)
            scatter(NCHUNK - 1, buf0)
        else:
            wait(NCHUNK - 2, buf0, sem0)
            issue(NCHUNK - 1, buf1, sem1)
            scatter(NCHUNK - 2, buf0)
            wait(NCHUNK - 1, buf1, sem1)
            scatter(NCHUNK - 1, buf1)
        plsc.subcore_barrier()

        pltpu.sync_copy(acc_sh.at[pl.ds(r0, ROWS_PER_TILE_INIT)],
                        psum.at[cid, pl.ds(r0, ROWS_PER_TILE_INIT)])

    fn = pl.kernel(
        body,
        out_type=jax.ShapeDtypeStruct((NC, S, D), jnp.float32),
        mesh=mesh,
        scratch_types=(
            pltpu.VMEM((G, K), jnp.int32),
            pltpu.VMEM((CHUNK, D), jnp.float32),
            pltpu.VMEM((CHUNK, D), jnp.float32),
            pltpu.VMEM_SHARED((S, D), jnp.float32),
            pltpu.SemaphoreType.DMA,
            pltpu.SemaphoreType.DMA,
            pltpu.SemaphoreType.DMA,
        ),
    )
    return fn(H_v, bidx, zacc)


def _hist_body(b_ref, out_ref):
    ids = b_ref[...]
    q = ids // 128
    r = ids - q * 128
    q_oh = (q == lax.broadcasted_iota(jnp.int32, (HB, 4), 1)
            ).astype(jnp.float32)
    r_oh = (r == lax.broadcasted_iota(jnp.int32, (HB, 128), 1)
            ).astype(jnp.float32)
    cmat = lax.dot_general(q_oh, r_oh, (((0,), (0,)), ((), ())),
                           preferred_element_type=jnp.float32)
    acc = jnp.concatenate([cmat, jnp.zeros((4, 128), jnp.float32)], axis=0)

    @pl.when(pl.program_id(0) == 0)
    def _init():
        out_ref[...] = jnp.zeros((8, 128), jnp.float32)

    out_ref[...] += acc


def _segment_counts(batch):
    bcol = batch.reshape(N, 1)
    hist = pl.pallas_call(
        _hist_body,
        grid=(HG,),
        in_specs=[pl.BlockSpec((HB, 1), lambda i: (i, 0))],
        out_specs=pl.BlockSpec((8, 128), lambda i: (0, 0)),
        out_shape=jax.ShapeDtypeStruct((8, 128), jnp.float32),
    )(bcol)
    return hist[0:4].reshape(S, 1)


def _finalize_body(ps_ref, pc_ref, out_ref):
    sums = ps_ref[0] + ps_ref[1]
    counts = jnp.maximum(pc_ref[...], 1.0)
    out_ref[...] = sums / counts


def kernel(H_v, batch):
    bidx = batch.reshape(NW, G, K)
    zacc = jnp.zeros((S, D), jnp.float32)
    psum = _sc_segment_sum(H_v, bidx, zacc)
    counts = _segment_counts(batch)
    return pl.pallas_call(
        _finalize_body,
        out_shape=jax.ShapeDtypeStruct((S, D), jnp.float32),
    )(psum, counts)


# speedup vs baseline: 1.3073x; 1.2005x over previous
import jax
import jax.numpy as jnp
from jax import lax
from jax.experimental import pallas as pl
from jax.experimental.pallas import tpu as pltpu
from jax.experimental.pallas import tpu_sc as plsc

N = 320000
D = 128
S = 512
NC = 2
NS = 16
NW = NC * NS
RPW = N // NW
K = 100
G = RPW // K
CHUNK = 200
GPC = CHUNK // K
NCHUNK = RPW // CHUNK
ROWS_PER_TILE_INIT = S // NS
HB = 8000
HG = N // HB


def _sc_segment_sum(H_v, bidx, zacc):
    mesh = plsc.VectorSubcoreMesh(core_axis_name="c", subcore_axis_name="s")

    def body(hv, idx_hbm, zacc_hbm, psum, idx_v, buf0, buf1, acc_sh,
             sem0, sem1, semS):
        cid = lax.axis_index("c")
        sid = lax.axis_index("s")
        wid = cid * NS + sid
        row_base = wid * RPW

        pltpu.sync_copy(idx_hbm.at[wid], idx_v)
        r0 = sid * ROWS_PER_TILE_INIT
        pltpu.sync_copy(zacc_hbm.at[pl.ds(r0, ROWS_PER_TILE_INIT)],
                        acc_sh.at[pl.ds(r0, ROWS_PER_TILE_INIT)])
        plsc.subcore_barrier()

        def hv_chunk(c):
            return hv.at[pl.ds(row_base + c * CHUNK, CHUNK)]

        def issue(c, buf, sem):
            pltpu.async_copy(hv_chunk(c), buf, sem)

        def wait(c, buf, sem):
            pltpu.make_async_copy(hv_chunk(c), buf, sem).wait()

        def scatter(c, buf):
            for g in range(GPC):
                jg = c * GPC + g
                pltpu.async_copy(buf.at[pl.ds(g * K, K)],
                                 acc_sh.at[idx_v.at[jg]], semS, add=True)
                pltpu.make_async_copy(
                    buf.at[pl.ds(g * K, K)], acc_sh.at[idx_v.at[jg]],
                    semS).wait()

        issue(0, buf0, sem0)

        def pair_body(j, carry):
            c0 = 2 * j
            wait(c0, buf0, sem0)
            issue(c0 + 1, buf1, sem1)
            scatter(c0, buf0)
            wait(c0 + 1, buf1, sem1)
            issue(c0 + 2, buf0, sem0)
            scatter(c0 + 1, buf1)
            return carry

        lax.fori_loop(0, (NCHUNK - 1) // 2, pair_body, 0)
        if NCHUNK % 2:
            wait(NCHUNK - 1, buf0, sem0)
            scatter(NCHUNK - 1, buf0)
        else:
            wait(NCHUNK - 2, buf0, sem0)
            issue(NCHUNK - 1, buf1, sem1)
            scatter(NCHUNK - 2, buf0)
            wait(NCHUNK - 1, buf1, sem1)
            scatter(NCHUNK - 1, buf1)
        plsc.subcore_barrier()

        pltpu.sync_copy(acc_sh.at[pl.ds(r0, ROWS_PER_TILE_INIT)],
                        psum.at[cid, pl.ds(r0, ROWS_PER_TILE_INIT)])

    fn = pl.kernel(
        body,
        out_type=jax.ShapeDtypeStruct((NC, S, D), jnp.float32),
        mesh=mesh,
        scratch_types=(
            pltpu.VMEM((G, K), jnp.int32),
            pltpu.VMEM((CHUNK, D), jnp.float32),
            pltpu.VMEM((CHUNK, D), jnp.float32),
            pltpu.VMEM_SHARED((S, D), jnp.float32),
            pltpu.SemaphoreType.DMA,
            pltpu.SemaphoreType.DMA,
            pltpu.SemaphoreType.DMA,
        ),
    )
    return fn(H_v, bidx, zacc)


def _hist_body(b_ref, out_ref):
    ids = b_ref[...]
    q = lax.shift_right_logical(ids, 7)
    r = lax.bitwise_and(ids, 127)
    q_oh = (q == lax.broadcasted_iota(jnp.int32, (HB, 4), 1)
            ).astype(jnp.bfloat16)
    r_oh = (r == lax.broadcasted_iota(jnp.int32, (HB, 128), 1)
            ).astype(jnp.bfloat16)
    cmat = lax.dot_general(q_oh, r_oh, (((0,), (0,)), ((), ())),
                           preferred_element_type=jnp.float32)
    acc = jnp.concatenate([cmat, jnp.zeros((4, 128), jnp.float32)], axis=0)

    @pl.when(pl.program_id(0) == 0)
    def _init():
        out_ref[...] = jnp.zeros((8, 128), jnp.float32)

    out_ref[...] += acc


def _segment_counts(batch):
    bcol = batch.reshape(N, 1)
    hist = pl.pallas_call(
        _hist_body,
        grid=(HG,),
        in_specs=[pl.BlockSpec((HB, 1), lambda i: (i, 0))],
        out_specs=pl.BlockSpec((8, 128), lambda i: (0, 0)),
        out_shape=jax.ShapeDtypeStruct((8, 128), jnp.float32),
    )(bcol)
    return hist[0:4].reshape(S, 1)


def _finalize_body(ps_ref, pc_ref, out_ref):
    sums = ps_ref[0] + ps_ref[1]
    counts = jnp.maximum(pc_ref[...], 1.0)
    out_ref[...] = sums / counts


def kernel(H_v, batch):
    bidx = batch.reshape(NW, G, K)
    zacc = jnp.zeros((S, D), jnp.float32)
    psum = _sc_segment_sum(H_v, bidx, zacc)
    counts = _segment_counts(batch)
    return pl.pallas_call(
        _finalize_body,
        out_shape=jax.ShapeDtypeStruct((S, D), jnp.float32),
    )(psum, counts)


# speedup vs baseline: 1.3184x; 1.0085x over previous
import jax
import jax.numpy as jnp
from jax import lax
from jax.experimental import pallas as pl
from jax.experimental.pallas import tpu as pltpu
from jax.experimental.pallas import tpu_sc as plsc

N = 320000
D = 128
S = 512
NC = 2
NS = 16
NW = NC * NS
RPW = N // NW
K = 100
G = RPW // K
CHUNK = 200
GPC = CHUNK // K
NCHUNK = RPW // CHUNK
ROWS_PER_TILE_INIT = S // NS
HB = 16000
HG = N // HB


def _sc_segment_sum(H_v, bidx, zacc):
    mesh = plsc.VectorSubcoreMesh(core_axis_name="c", subcore_axis_name="s")

    def body(hv, idx_hbm, zacc_hbm, psum, idx_v, buf0, buf1, acc_sh,
             sem0, sem1, semS):
        cid = lax.axis_index("c")
        sid = lax.axis_index("s")
        wid = cid * NS + sid
        row_base = wid * RPW

        pltpu.sync_copy(idx_hbm.at[wid], idx_v)
        r0 = sid * ROWS_PER_TILE_INIT
        pltpu.sync_copy(zacc_hbm.at[pl.ds(r0, ROWS_PER_TILE_INIT)],
                        acc_sh.at[pl.ds(r0, ROWS_PER_TILE_INIT)])
        plsc.subcore_barrier()

        def hv_chunk(c):
            return hv.at[pl.ds(row_base + c * CHUNK, CHUNK)]

        def issue(c, buf, sem):
            pltpu.async_copy(hv_chunk(c), buf, sem)

        def wait(c, buf, sem):
            pltpu.make_async_copy(hv_chunk(c), buf, sem).wait()

        def scatter(c, buf):
            for g in range(GPC):
                jg = c * GPC + g
                pltpu.async_copy(buf.at[pl.ds(g * K, K)],
                                 acc_sh.at[idx_v.at[jg]], semS, add=True)
                pltpu.make_async_copy(
                    buf.at[pl.ds(g * K, K)], acc_sh.at[idx_v.at[jg]],
                    semS).wait()

        issue(0, buf0, sem0)

        def pair_body(j, carry):
            c0 = 2 * j
            wait(c0, buf0, sem0)
            issue(c0 + 1, buf1, sem1)
            scatter(c0, buf0)
            wait(c0 + 1, buf1, sem1)
            issue(c0 + 2, buf0, sem0)
            scatter(c0 + 1, buf1)
            return carry

        lax.fori_loop(0, (NCHUNK - 1) // 2, pair_body, 0)
        if NCHUNK % 2:
            wait(NCHUNK - 1, buf0, sem0)
            scatter(NCHUNK - 1, buf0)
        else:
            wait(NCHUNK - 2, buf0, sem0)
            issue(NCHUNK - 1, buf1, sem1)
            scatter(NCHUNK - 2, buf0)
            wait(NCHUNK - 1, buf1, sem1)
            scatter(NCHUNK - 1, buf1)
        plsc.subcore_barrier()

        pltpu.sync_copy(acc_sh.at[pl.ds(r0, ROWS_PER_TILE_INIT)],
                        psum.at[cid, pl.ds(r0, ROWS_PER_TILE_INIT)])

    fn = pl.kernel(
        body,
        out_type=jax.ShapeDtypeStruct((NC, S, D), jnp.float32),
        mesh=mesh,
        scratch_types=(
            pltpu.VMEM((G, K), jnp.int32),
            pltpu.VMEM((CHUNK, D), jnp.float32),
            pltpu.VMEM((CHUNK, D), jnp.float32),
            pltpu.VMEM_SHARED((S, D), jnp.float32),
            pltpu.SemaphoreType.DMA,
            pltpu.SemaphoreType.DMA,
            pltpu.SemaphoreType.DMA,
        ),
    )
    return fn(H_v, bidx, zacc)


def _hist_body(b_ref, out_ref):
    ids = b_ref[...]
    q = lax.shift_right_logical(ids, 7)
    r = lax.bitwise_and(ids, 127)
    q_oh = (q == lax.broadcasted_iota(jnp.int32, (HB, 4), 1)
            ).astype(jnp.bfloat16)
    r_oh = (r == lax.broadcasted_iota(jnp.int32, (HB, 128), 1)
            ).astype(jnp.bfloat16)
    cmat = lax.dot_general(q_oh, r_oh, (((0,), (0,)), ((), ())),
                           preferred_element_type=jnp.float32)
    acc = jnp.concatenate([cmat, jnp.zeros((4, 128), jnp.float32)], axis=0)

    @pl.when(pl.program_id(0) == 0)
    def _init():
        out_ref[...] = jnp.zeros((8, 128), jnp.float32)

    out_ref[...] += acc


def _segment_counts(batch):
    bcol = batch.reshape(N, 1)
    hist = pl.pallas_call(
        _hist_body,
        grid=(HG,),
        in_specs=[pl.BlockSpec((HB, 1), lambda i: (i, 0))],
        out_specs=pl.BlockSpec((8, 128), lambda i: (0, 0)),
        out_shape=jax.ShapeDtypeStruct((8, 128), jnp.float32),
    )(bcol)
    return hist[0:4].reshape(S, 1)


def _finalize_body(ps_ref, pc_ref, out_ref):
    sums = ps_ref[0] + ps_ref[1]
    counts = jnp.maximum(pc_ref[...], 1.0)
    out_ref[...] = sums / counts


def kernel(H_v, batch):
    bidx = batch.reshape(NW, G, K)
    zacc = jnp.zeros((S, D), jnp.float32)
    psum = _sc_segment_sum(H_v, bidx, zacc)
    counts = _segment_counts(batch)
    return pl.pallas_call(
        _finalize_body,
        out_shape=jax.ShapeDtypeStruct((S, D), jnp.float32),
    )(psum, counts)
